# baseline (speedup 1.0000x reference)
import functools

import jax
import jax.numpy as jnp
from jax import lax
from jax.experimental import pallas as pl
from jax.experimental.pallas import tpu as pltpu
from jax.experimental.pallas import tpu_sc as plsc

N = 10000
E = 320000
G = 2
D = 128
H = 64
C = 8
S = 5
LP_LAYERS = 3

NC = 2
NS = 16
L = 16
CH = 80
BN = 1000

_i32 = jnp.int32
_f32 = jnp.float32


def _iota16():
  return lax.iota(_i32, L)


def _full16(v):
  return jnp.full((L,), v, _i32)


def _tc1_body(f0_ref, f1_ref, w0_ref, b0_ref, w1_ref, b1_ref, acat_ref,
              h0_ref, h1_ref, es_ref):
  f0 = f0_ref[...]
  h0_ref[...] = jnp.dot(f0, w0_ref[...],
                        preferred_element_type=_f32) + b0_ref[...]
  h1_ref[...] = jnp.dot(f1_ref[...], w1_ref[...],
                        preferred_element_type=_f32) + b1_ref[...]
  es_ref[...] = lax.dot_general(f0, acat_ref[...],
                                (((1,), (1,)), ((), ())),
                                preferred_element_type=_f32)


def _tc1(feats0, feats1, w0, b0, w1, b1, acat):
  grid = N // BN
  return pl.pallas_call(
      _tc1_body,
      grid=(grid,),
      in_specs=[
          pl.BlockSpec((BN, D), lambda i: (i, 0)),
          pl.BlockSpec((BN, D), lambda i: (i, 0)),
          pl.BlockSpec((D, H), lambda i: (0, 0)),
          pl.BlockSpec((1, H), lambda i: (0, 0)),
          pl.BlockSpec((D, H), lambda i: (0, 0)),
          pl.BlockSpec((1, H), lambda i: (0, 0)),
          pl.BlockSpec((2 * G, D), lambda i: (0, 0)),
      ],
      out_specs=[
          pl.BlockSpec((BN, H), lambda i: (i, 0)),
          pl.BlockSpec((BN, H), lambda i: (i, 0)),
          pl.BlockSpec((BN, 2 * G), lambda i: (i, 0)),
      ],
      out_shape=[
          jax.ShapeDtypeStruct((N, H), _f32),
          jax.ShapeDtypeStruct((N, H), _f32),
          jax.ShapeDtypeStruct((N, 2 * G), _f32),
      ],
  )(feats0, feats1, w0, b0.reshape(1, H), w1, b1.reshape(1, H), acat)


def _tc2_body(h0_ref, nei_ref, wo_ref, bo_ref, lp0_ref, lp1_ref, att_ref,
              al_ref, lg_ref, lp_ref, ns_ref):
  z = h0_ref[...] + nei_ref[...]
  z = jnp.where(z > 0, z, jnp.exp(z) - 1.0)
  ns = jnp.dot(z, wo_ref[...], preferred_element_type=_f32) + bo_ref[...]
  a0 = att_ref[:, 0:1]
  a1 = att_ref[:, 1:2]
  m = jnp.maximum(a0, a1)
  e0 = jnp.exp(a0 - m)
  e1 = jnp.exp(a1 - m)
  inv = 1.0 / (e0 + e1)
  lp = (e0 * inv) * lp0_ref[...] + (e1 * inv) * lp1_ref[...]
  al = al_ref[...]
  sa = 1.0 / (1.0 + jnp.exp(-al))
  sb = 1.0 / (1.0 + jnp.exp(al))
  lg_ref[...] = sa * lp + sb * ns
  lp_ref[...] = lp
  ns_ref[...] = ns


def _tc2(h0, nei, wo, bo, lp0, lp1, att2, alpha):
  grid = N // BN
  return pl.pallas_call(
      _tc2_body,
      grid=(grid,),
      in_specs=[
          pl.BlockSpec((BN, H), lambda i: (i, 0)),
          pl.BlockSpec((BN, H), lambda i: (i, 0)),
          pl.BlockSpec((H, C), lambda i: (0, 0)),
          pl.BlockSpec((1, C), lambda i: (0, 0)),
          pl.BlockSpec((BN, C), lambda i: (i, 0)),
          pl.BlockSpec((BN, C), lambda i: (i, 0)),
          pl.BlockSpec((BN, G), lambda i: (i, 0)),
          pl.BlockSpec((BN, 1), lambda i: (i, 0)),
      ],
      out_specs=[
          pl.BlockSpec((BN, C), lambda i: (i, 0)),
          pl.BlockSpec((BN, C), lambda i: (i, 0)),
          pl.BlockSpec((BN, C), lambda i: (i, 0)),
      ],
      out_shape=[
          jax.ShapeDtypeStruct((N, C), _f32),
          jax.ShapeDtypeStruct((N, C), _f32),
          jax.ShapeDtypeStruct((N, C), _f32),
      ],
  )(h0, nei, wo, bo.reshape(1, C), lp0, lp1, att2, alpha)


EPT = E // NS
CHL = 4000
KPB = CHL // CH
NBIG = EPT // CHL
NP = 10240
TPR = NP // NS
EPR = 80
NGW = 8
ZR = 40
NGRP = N // L


def _lp_body(edges_ref, esed_ref, h_in_ref, maskf_ref, lo_ref, h1_ref,
             nif_ref, h_out_ref, nei_ref,
             acc_sh, es_t, ed_t, htab, srcb0, dstb0, srcb1, dstb1, cb0, cb1,
             zbuf, ebuf, mbuf, lobuf, obuf, nidx, gbuf, obuf64, semz, semd,
             semd2, sem0, sem1):
  g = lax.axis_index("c")
  t = lax.axis_index("s")

  pltpu.sync_copy(esed_ref.at[g, 0], es_t)
  pltpu.sync_copy(esed_ref.at[g, 1], ed_t)

  zeros = jnp.zeros((L,), _f32)
  rowoff = lax.shift_right_logical(_iota16(), 3)
  coloff = lax.bitwise_and(_iota16(), _full16(C - 1))
  for i in range(ZR // 2):
    plsc.store_scatter(zbuf, [rowoff + (2 * i), coloff], zeros)

  def zero_own_rows(rb):
    zdescs = [
        pltpu.make_async_copy(zbuf, acc_sh.at[pl.ds(rb + ZR * k, ZR)], semz)
        for k in range(EPR // ZR)
    ]
    for d in zdescs:
      d.start()
    for d in zdescs:
      d.wait()

  for pss in range(TPR // EPR):
    zero_own_rows(t * TPR + pss * EPR)

  cbs = [cb0, cb1]
  sems = [sem0, sem1]

  def layer_once(li):
    @pl.when(li == 0)
    def _():
      pltpu.sync_copy(h_in_ref.at[pl.ds(0, N)], htab)

    @pl.when(li > 0)
    def _():
      pltpu.sync_copy(h_out_ref.at[g, pl.ds(0, N)], htab)

    plsc.subcore_barrier()

    lbufs = [(srcb0, dstb0), (srcb1, dstb1)]

    def fire_block(b, sbuf, dbuf, sem):
      off_s = g * (2 * E) + t * EPT + b * CHL
      off_d = off_s + E
      ldescs = [pltpu.make_async_copy(edges_ref.at[pl.ds(off_s, CHL)], sbuf,
                                      sem)]
      ldescs += [
          pltpu.make_async_copy(edges_ref.at[pl.ds(off_d + CH * k, CH)],
                                dbuf.at[k], sem)
          for k in range(KPB)
      ]
      for d in ldescs:
        d.start()
      return ldescs

    def compute_block(sbuf, dbuf):
      def compute_chunk(kdyn, cb):
        for j in range(CH // L):
          rows = _iota16() + (16 * j)
          s16 = sbuf[pl.ds(kdyn * CH + L * j, L)]
          d16 = dbuf[kdyn, pl.ds(L * j, L)]
          es = plsc.load_gather(es_t, [s16])
          ed = plsc.load_gather(ed_t, [d16])
          e = es + ed
          e = jnp.maximum(e, 0.2 * e)
          ex = jnp.exp(e)
          for c in range(C):
            hv = plsc.load_gather(htab, [s16, _full16(c)])
            plsc.store_scatter(cb, [rows, _full16(c)], ex * hv)

      def pair_body(i, carry):
        for half in range(2):
          k = 2 * i + half
          p = half

          @pl.when(i > 0)
          def _():
            pltpu.make_async_copy(cbs[p], acc_sh.at[dbuf.at[k - 2]],
                                  sems[p]).wait()

          compute_chunk(k, cbs[p])
          pltpu.make_async_copy(cbs[p], acc_sh.at[dbuf.at[k]],
                                sems[p]).start(add=True)
        return carry

      lax.fori_loop(0, KPB // 2, pair_body, 0)
      for p in range(2):
        pltpu.make_async_copy(cbs[p], acc_sh.at[dbuf.at[KPB - 2 + p]],
                              sems[p]).wait()

    for d in fire_block(0, *lbufs[0], semd):
      d.wait()

    def blockpair_body(i, carry):
      b0 = 2 * i
      fired_b = fire_block(b0 + 1, *lbufs[1], semd2)
      compute_block(*lbufs[0])
      fired_a = fire_block(b0 + 2, *lbufs[0], semd)
      for d in fired_b:
        d.wait()
      compute_block(*lbufs[1])
      for d in fired_a:
        d.wait()
      return carry

    lax.fori_loop(0, NBIG // 2, blockpair_body, 0)
    compute_block(*lbufs[0])
    plsc.subcore_barrier()

    for pss in range(TPR // EPR):
      rb = t * TPR + pss * EPR
      pltpu.sync_copy(acc_sh.at[pl.ds(rb, EPR)], ebuf)
      pltpu.sync_copy(maskf_ref.at[pl.ds(rb, EPR)], mbuf)
      pltpu.sync_copy(lo_ref.at[pl.ds(rb, EPR)], lobuf)

      def ep_grp(grp, carry):
        rows = _iota16() + (L * grp)
        a = [plsc.load_gather(ebuf, [rows, _full16(c)]) for c in range(C)]
        s = a[0]
        for c in range(1, C):
          s = s + a[c]
        m = plsc.load_gather(mbuf, [rows])
        w = (1.0 - m) / (s + 1e-16)
        for c in range(C):
          lo = plsc.load_gather(lobuf, [rows, _full16(c)])
          plsc.store_scatter(obuf, [rows, _full16(c)], a[c] * w + lo * m)
        return carry

      lax.fori_loop(0, EPR // L, ep_grp, 0)
      pltpu.sync_copy(obuf, h_out_ref.at[g, pl.ds(rb, EPR)])
      zero_own_rows(rb)
    plsc.subcore_barrier()

  def layers_body(li, carry):
    layer_once(li)
    return carry

  lax.fori_loop(0, LP_LAYERS, layers_body, 0)

  w = lax.axis_index("s") * NC + g

  def grp_body(j, carry):
    gi = w + NC * NS * j

    @pl.when(gi < N // NGW)
    def _():
      pltpu.sync_copy(nif_ref.at[pl.ds(S * NGW * gi, S * NGW)], nidx)
      pltpu.async_copy(h1_ref.at[nidx], gbuf, semd).wait()
      for i in range(NGW):
        for cg in range(H // L):
          acc = gbuf[S * i, pl.ds(L * cg, L)]
          for kk in range(1, S):
            acc = acc + gbuf[S * i + kk, pl.ds(L * cg, L)]
          obuf64[i, pl.ds(L * cg, L)] = acc * (1.0 / S)
      pltpu.sync_copy(obuf64, nei_ref.at[pl.ds(NGW * gi, NGW)])

    return carry

  lax.fori_loop(0, (N // NGW + NC * NS - 1) // (NC * NS), grp_body, 0)


def _make_lp():
  mesh = plsc.VectorSubcoreMesh(core_axis_name="c", subcore_axis_name="s",
                                num_cores=NC, num_subcores=NS)
  return pl.kernel(
      _lp_body,
      out_type=(jax.ShapeDtypeStruct((G, NP, C), _f32),
                jax.ShapeDtypeStruct((N, H), _f32)),
      mesh=mesh,
      compiler_params=pltpu.CompilerParams(needs_layout_passes=False,
                                           use_tc_tiling_on_sc=False),
      scratch_types=[
          pltpu.VMEM_SHARED((NP, C), _f32),
          pltpu.VMEM((N,), _f32),
          pltpu.VMEM((N,), _f32),
          pltpu.VMEM((N, C), _f32),
          pltpu.VMEM((CHL,), _i32),
          pltpu.VMEM((KPB, CH), _i32),
          pltpu.VMEM((CHL,), _i32),
          pltpu.VMEM((KPB, CH), _i32),
          pltpu.VMEM((CH, C), _f32),
          pltpu.VMEM((CH, C), _f32),
          pltpu.VMEM((ZR, C), _f32),
          pltpu.VMEM((EPR, C), _f32),
          pltpu.VMEM((EPR,), _f32),
          pltpu.VMEM((EPR, C), _f32),
          pltpu.VMEM((EPR, C), _f32),
          pltpu.VMEM((S * NGW,), _i32),
          pltpu.VMEM((S * NGW, H), _f32),
          pltpu.VMEM((NGW, H), _f32),
          pltpu.SemaphoreType.DMA,
          pltpu.SemaphoreType.DMA,
          pltpu.SemaphoreType.DMA,
          pltpu.SemaphoreType.DMA,
          pltpu.SemaphoreType.DMA,
      ],
  )


def kernel(feats0, feats1, label_init, labels_one_hot, byte_idx_train,
           edge_index, nei_idx, alpha, attention,
           a_src, a_dst, fc0_W, fc0_b, fc1_W, fc1_b, W_out, b_out):
  maskf = byte_idx_train.astype(_f32).reshape(N)
  acat = jnp.stack([a_src[0], a_dst[0], a_src[1], a_dst[1]])

  h0, h1, esed4 = _tc1(feats0, feats1, fc0_W, fc0_b, fc1_W, fc1_b, acat)
  esed = esed4.T.reshape(G, 2, N)

  lp = _make_lp()
  edges = edge_index.astype(_i32).reshape(G * 2 * E)
  pad = ((0, NP - N), (0, 0))
  maskf_p = jnp.pad(maskf.reshape(N, 1), pad).reshape(NP)
  lo_p = jnp.pad(labels_one_hot, pad)
  h_init = jnp.pad(label_init, pad)
  h_st, nei = lp(edges, esed, h_init, maskf_p, lo_p, h1,
                 nei_idx.astype(_i32).reshape(N * S))

  logits, logits_lp, logits_ns = _tc2(
      h0, nei, W_out, b_out, h_st[0, :N], h_st[1, :N],
      attention.reshape(N, G), alpha)
  return (logits, logits_lp, logits_ns)


# speedup vs baseline: 1.0188x; 1.0188x over previous
import functools

import jax
import jax.numpy as jnp
from jax import lax
from jax.experimental import pallas as pl
from jax.experimental.pallas import tpu as pltpu
from jax.experimental.pallas import tpu_sc as plsc

N = 10000
E = 320000
G = 2
D = 128
H = 64
C = 8
S = 5
LP_LAYERS = 3

NC = 2
NS = 16
L = 16
CH = 80
BN = 1000

_i32 = jnp.int32
_f32 = jnp.float32


def _iota16():
  return lax.iota(_i32, L)


def _full16(v):
  return jnp.full((L,), v, _i32)


def _tc1_body(f0_ref, f1_ref, w0_ref, b0_ref, w1_ref, b1_ref, acat_ref,
              h0_ref, h1_ref, es_ref):
  f0 = f0_ref[...]
  h0_ref[...] = jnp.dot(f0, w0_ref[...],
                        preferred_element_type=_f32) + b0_ref[...]
  h1_ref[...] = jnp.dot(f1_ref[...], w1_ref[...],
                        preferred_element_type=_f32) + b1_ref[...]
  es_ref[...] = lax.dot_general(f0, acat_ref[...],
                                (((1,), (1,)), ((), ())),
                                preferred_element_type=_f32)


def _tc1(feats0, feats1, w0, b0, w1, b1, acat):
  grid = N // BN
  return pl.pallas_call(
      _tc1_body,
      grid=(grid,),
      in_specs=[
          pl.BlockSpec((BN, D), lambda i: (i, 0)),
          pl.BlockSpec((BN, D), lambda i: (i, 0)),
          pl.BlockSpec((D, H), lambda i: (0, 0)),
          pl.BlockSpec((1, H), lambda i: (0, 0)),
          pl.BlockSpec((D, H), lambda i: (0, 0)),
          pl.BlockSpec((1, H), lambda i: (0, 0)),
          pl.BlockSpec((2 * G, D), lambda i: (0, 0)),
      ],
      out_specs=[
          pl.BlockSpec((BN, H), lambda i: (i, 0)),
          pl.BlockSpec((BN, H), lambda i: (i, 0)),
          pl.BlockSpec((BN, 2 * G), lambda i: (i, 0)),
      ],
      out_shape=[
          jax.ShapeDtypeStruct((N, H), _f32),
          jax.ShapeDtypeStruct((N, H), _f32),
          jax.ShapeDtypeStruct((N, 2 * G), _f32),
      ],
  )(feats0, feats1, w0, b0.reshape(1, H), w1, b1.reshape(1, H), acat)


def _tc2_body(h0_ref, nei_ref, wo_ref, bo_ref, lp0_ref, lp1_ref, att_ref,
              al_ref, lg_ref, lp_ref, ns_ref):
  z = h0_ref[...] + nei_ref[...]
  z = jnp.where(z > 0, z, jnp.exp(z) - 1.0)
  ns = jnp.dot(z, wo_ref[...], preferred_element_type=_f32) + bo_ref[...]
  a0 = att_ref[:, 0:1]
  a1 = att_ref[:, 1:2]
  m = jnp.maximum(a0, a1)
  e0 = jnp.exp(a0 - m)
  e1 = jnp.exp(a1 - m)
  inv = 1.0 / (e0 + e1)
  lp = (e0 * inv) * lp0_ref[...] + (e1 * inv) * lp1_ref[...]
  al = al_ref[...]
  sa = 1.0 / (1.0 + jnp.exp(-al))
  sb = 1.0 / (1.0 + jnp.exp(al))
  lg_ref[...] = sa * lp + sb * ns
  lp_ref[...] = lp
  ns_ref[...] = ns


def _tc2(h0, nei, wo, bo, lp0, lp1, att2, alpha):
  grid = N // BN
  return pl.pallas_call(
      _tc2_body,
      grid=(grid,),
      in_specs=[
          pl.BlockSpec((BN, H), lambda i: (i, 0)),
          pl.BlockSpec((BN, H), lambda i: (i, 0)),
          pl.BlockSpec((H, C), lambda i: (0, 0)),
          pl.BlockSpec((1, C), lambda i: (0, 0)),
          pl.BlockSpec((BN, C), lambda i: (i, 0)),
          pl.BlockSpec((BN, C), lambda i: (i, 0)),
          pl.BlockSpec((BN, G), lambda i: (i, 0)),
          pl.BlockSpec((BN, 1), lambda i: (i, 0)),
      ],
      out_specs=[
          pl.BlockSpec((BN, C), lambda i: (i, 0)),
          pl.BlockSpec((BN, C), lambda i: (i, 0)),
          pl.BlockSpec((BN, C), lambda i: (i, 0)),
      ],
      out_shape=[
          jax.ShapeDtypeStruct((N, C), _f32),
          jax.ShapeDtypeStruct((N, C), _f32),
          jax.ShapeDtypeStruct((N, C), _f32),
      ],
  )(h0, nei, wo, bo.reshape(1, C), lp0, lp1, att2, alpha)


EPT = E // NS
CHL = 4000
KPB = CHL // CH
NBIG = EPT // CHL
NP = 10240
TPR = NP // NS
EPR = 160
NGW = 8
ZR = 80
NGRP = N // L


def _lp_body(edges_ref, esed_ref, h_in_ref, maskf_ref, lo_ref, h1_ref,
             nif_ref, h_out_ref, nei_ref,
             acc_sh, es_t, ed_t, htab, srcb0, dstb0, cb0, cb1,
             zbuf, ebuf, mbuf, lobuf, obuf, nidx, gbuf, obuf64, semz, semd,
             sem0, sem1):
  g = lax.axis_index("c")
  t = lax.axis_index("s")

  pltpu.sync_copy(esed_ref.at[g, 0], es_t)
  pltpu.sync_copy(esed_ref.at[g, 1], ed_t)

  zeros = jnp.zeros((L,), _f32)
  rowoff = lax.shift_right_logical(_iota16(), 3)
  coloff = lax.bitwise_and(_iota16(), _full16(C - 1))
  for i in range(ZR // 2):
    plsc.store_scatter(zbuf, [rowoff + (2 * i), coloff], zeros)

  def zero_own_rows(rb):
    zdescs = [
        pltpu.make_async_copy(zbuf, acc_sh.at[pl.ds(rb + ZR * k, ZR)], semz)
        for k in range(EPR // ZR)
    ]
    for d in zdescs:
      d.start()
    for d in zdescs:
      d.wait()

  for pss in range(TPR // EPR):
    zero_own_rows(t * TPR + pss * EPR)

  cbs = [cb0, cb1]
  sems = [sem0, sem1]

  def layer_once(li):
    @pl.when(li == 0)
    def _():
      pltpu.sync_copy(h_in_ref.at[pl.ds(0, N)], htab)

    @pl.when(li > 0)
    def _():
      pltpu.sync_copy(h_out_ref.at[g, pl.ds(0, N)], htab)

    plsc.subcore_barrier()

    def fire_block(b, sbuf, dbuf, sem):
      off_s = g * (2 * E) + t * EPT + b * CHL
      off_d = off_s + E
      ldescs = [pltpu.make_async_copy(edges_ref.at[pl.ds(off_s, CHL)], sbuf,
                                      sem)]
      ldescs += [
          pltpu.make_async_copy(edges_ref.at[pl.ds(off_d + CH * k, CH)],
                                dbuf.at[k], sem)
          for k in range(KPB)
      ]
      for d in ldescs:
        d.start()
      return ldescs

    def compute_block(sbuf, dbuf):
      def compute_chunk(kdyn, cb):
        for j in range(CH // L):
          rows = _iota16() + (16 * j)
          s16 = sbuf[pl.ds(kdyn * CH + L * j, L)]
          d16 = dbuf[kdyn, pl.ds(L * j, L)]
          es = plsc.load_gather(es_t, [s16])
          ed = plsc.load_gather(ed_t, [d16])
          e = es + ed
          e = jnp.maximum(e, 0.2 * e)
          ex = jnp.exp(e)
          for c in range(C):
            hv = plsc.load_gather(htab, [s16, _full16(c)])
            plsc.store_scatter(cb, [rows, _full16(c)], ex * hv)

      def pair_body(i, carry):
        for half in range(2):
          k = 2 * i + half
          p = half

          @pl.when(i > 0)
          def _():
            pltpu.make_async_copy(cbs[p], acc_sh.at[dbuf.at[k - 2]],
                                  sems[p]).wait()

          compute_chunk(k, cbs[p])
          pltpu.make_async_copy(cbs[p], acc_sh.at[dbuf.at[k]],
                                sems[p]).start(add=True)
        return carry

      lax.fori_loop(0, KPB // 2, pair_body, 0)
      for p in range(2):
        pltpu.make_async_copy(cbs[p], acc_sh.at[dbuf.at[KPB - 2 + p]],
                              sems[p]).wait()

    def big_body(b, carry):
      for d in fire_block(b, srcb0, dstb0, semd):
        d.wait()
      compute_block(srcb0, dstb0)
      return carry

    lax.fori_loop(0, NBIG, big_body, 0)
    plsc.subcore_barrier()

    for pss in range(TPR // EPR):
      rb = t * TPR + pss * EPR
      pltpu.sync_copy(acc_sh.at[pl.ds(rb, EPR)], ebuf)
      pltpu.sync_copy(maskf_ref.at[pl.ds(rb, EPR)], mbuf)
      pltpu.sync_copy(lo_ref.at[pl.ds(rb, EPR)], lobuf)

      def ep_grp(grp, carry):
        rows = _iota16() + (L * grp)
        a = [plsc.load_gather(ebuf, [rows, _full16(c)]) for c in range(C)]
        s = a[0]
        for c in range(1, C):
          s = s + a[c]
        m = plsc.load_gather(mbuf, [rows])
        w = (1.0 - m) / (s + 1e-16)
        for c in range(C):
          lo = plsc.load_gather(lobuf, [rows, _full16(c)])
          plsc.store_scatter(obuf, [rows, _full16(c)], a[c] * w + lo * m)
        return carry

      lax.fori_loop(0, EPR // L, ep_grp, 0)
      pltpu.sync_copy(obuf, h_out_ref.at[g, pl.ds(rb, EPR)])
      zero_own_rows(rb)
    plsc.subcore_barrier()

  def layers_body(li, carry):
    layer_once(li)
    return carry

  lax.fori_loop(0, LP_LAYERS, layers_body, 0)

  w = lax.axis_index("s") * NC + g

  def grp_body(j, carry):
    gi = w + NC * NS * j

    @pl.when(gi < N // NGW)
    def _():
      pltpu.sync_copy(nif_ref.at[pl.ds(S * NGW * gi, S * NGW)], nidx)
      pltpu.async_copy(h1_ref.at[nidx], gbuf, semd).wait()
      for i in range(NGW):
        for cg in range(H // L):
          acc = gbuf[S * i, pl.ds(L * cg, L)]
          for kk in range(1, S):
            acc = acc + gbuf[S * i + kk, pl.ds(L * cg, L)]
          obuf64[i, pl.ds(L * cg, L)] = acc * (1.0 / S)
      pltpu.sync_copy(obuf64, nei_ref.at[pl.ds(NGW * gi, NGW)])

    return carry

  lax.fori_loop(0, (N // NGW + NC * NS - 1) // (NC * NS), grp_body, 0)


def _make_lp():
  mesh = plsc.VectorSubcoreMesh(core_axis_name="c", subcore_axis_name="s",
                                num_cores=NC, num_subcores=NS)
  return pl.kernel(
      _lp_body,
      out_type=(jax.ShapeDtypeStruct((G, NP, C), _f32),
                jax.ShapeDtypeStruct((N, H), _f32)),
      mesh=mesh,
      compiler_params=pltpu.CompilerParams(needs_layout_passes=False,
                                           use_tc_tiling_on_sc=False),
      scratch_types=[
          pltpu.VMEM_SHARED((NP, C), _f32),
          pltpu.VMEM((N,), _f32),
          pltpu.VMEM((N,), _f32),
          pltpu.VMEM((N, C), _f32),
          pltpu.VMEM((CHL,), _i32),
          pltpu.VMEM((KPB, CH), _i32),
          pltpu.VMEM((CH, C), _f32),
          pltpu.VMEM((CH, C), _f32),
          pltpu.VMEM((ZR, C), _f32),
          pltpu.VMEM((EPR, C), _f32),
          pltpu.VMEM((EPR,), _f32),
          pltpu.VMEM((EPR, C), _f32),
          pltpu.VMEM((EPR, C), _f32),
          pltpu.VMEM((S * NGW,), _i32),
          pltpu.VMEM((S * NGW, H), _f32),
          pltpu.VMEM((NGW, H), _f32),
          pltpu.SemaphoreType.DMA,
          pltpu.SemaphoreType.DMA,
          pltpu.SemaphoreType.DMA,
          pltpu.SemaphoreType.DMA,
      ],
  )


def kernel(feats0, feats1, label_init, labels_one_hot, byte_idx_train,
           edge_index, nei_idx, alpha, attention,
           a_src, a_dst, fc0_W, fc0_b, fc1_W, fc1_b, W_out, b_out):
  maskf = byte_idx_train.astype(_f32).reshape(N)
  acat = jnp.stack([a_src[0], a_dst[0], a_src[1], a_dst[1]])

  h0, h1, esed4 = _tc1(feats0, feats1, fc0_W, fc0_b, fc1_W, fc1_b, acat)
  esed = esed4.T.reshape(G, 2, N)

  lp = _make_lp()
  edges = edge_index.astype(_i32).reshape(G * 2 * E)
  pad = ((0, NP - N), (0, 0))
  maskf_p = jnp.pad(maskf.reshape(N, 1), pad).reshape(NP)
  lo_p = jnp.pad(labels_one_hot, pad)
  h_init = jnp.pad(label_init, pad)
  h_st, nei = lp(edges, esed, h_init, maskf_p, lo_p, h1,
                 nei_idx.astype(_i32).reshape(N * S))

  logits, logits_lp, logits_ns = _tc2(
      h0, nei, W_out, b_out, h_st[0, :N], h_st[1, :N],
      attention.reshape(N, G), alpha)
  return (logits, logits_lp, logits_ns)
